# initial kernel scaffold (unmeasured)
import jax
import jax.numpy as jnp
from jax import lax
from jax.experimental import pallas as pl
from jax.experimental.pallas import tpu as pltpu


def kernel(
    x,
):
    def body(*refs):
        pass

    out_shape = jax.ShapeDtypeStruct(..., jnp.float32)
    return pl.pallas_call(body, out_shape=out_shape)(...)


# baseline (device time: 12541 ns/iter reference)
import jax
import jax.numpy as jnp
from jax import lax
from jax.experimental import pallas as pl
from jax.experimental.pallas import tpu as pltpu

N_DEV = 32


def kernel(x):
    m_per, n = x.shape
    total_rows = N_DEV * m_per

    def body(x_ref, out_ref, acc_ref, send_sems, recv_sems):
        my_i = lax.axis_index("i")

        barrier = pltpu.get_barrier_semaphore()
        for k in range(1, N_DEV):
            pl.semaphore_signal(
                barrier,
                inc=1,
                device_id=((my_i + k) % N_DEV,),
                device_id_type=pl.DeviceIdType.MESH,
            )
        pl.semaphore_wait(barrier, N_DEV - 1)

        acc_ref[pl.ds(0, 1), :] = jnp.sum(
            x_ref[...].astype(jnp.float32), axis=0, keepdims=True
        )

        sends = []
        for k in range(1, N_DEV):
            rdma = pltpu.make_async_remote_copy(
                src_ref=acc_ref.at[pl.ds(0, 1)],
                dst_ref=acc_ref.at[pl.ds(N_DEV - k, 1)],
                send_sem=send_sems.at[k],
                recv_sem=recv_sems.at[N_DEV - k],
                device_id=((my_i + k) % N_DEV,),
                device_id_type=pl.DeviceIdType.MESH,
            )
            rdma.start()
            sends.append(rdma)

        for j in range(1, N_DEV):
            recv = pltpu.make_async_remote_copy(
                src_ref=acc_ref.at[pl.ds(0, 1)],
                dst_ref=acc_ref.at[pl.ds(j, 1)],
                send_sem=send_sems.at[0],
                recv_sem=recv_sems.at[j],
                device_id=((my_i + 1) % N_DEV,),
                device_id_type=pl.DeviceIdType.MESH,
            )
            recv.wait_recv()

        out_ref[...] = (
            jnp.sum(acc_ref[...], axis=0, keepdims=True) * (1.0 / total_rows)
        ).astype(jnp.float32)

        for rdma in sends:
            rdma.wait_send()

    return pl.pallas_call(
        body,
        out_shape=jax.ShapeDtypeStruct((1, n), jnp.float32),
        in_specs=[pl.BlockSpec(memory_space=pltpu.VMEM)],
        out_specs=pl.BlockSpec(memory_space=pltpu.VMEM),
        scratch_shapes=[
            pltpu.VMEM((N_DEV, n), jnp.float32),
            pltpu.SemaphoreType.DMA((N_DEV,)),
            pltpu.SemaphoreType.DMA((N_DEV,)),
        ],
        compiler_params=pltpu.CompilerParams(collective_id=0),
    )(x)


# device time: 12449 ns/iter; 1.0074x vs baseline; 1.0074x over previous
import jax
import jax.numpy as jnp
from jax import lax
from jax.experimental import pallas as pl
from jax.experimental.pallas import tpu as pltpu

N_DEV = 32


def kernel(x):
    m_per, n = x.shape
    total_rows = N_DEV * m_per

    def body(x_ref, out_ref, acc_ref, send_sems, recv_sems):
        my_i = lax.axis_index("i")

        barrier = pltpu.get_barrier_semaphore()
        for k in range(1, N_DEV):
            pl.semaphore_signal(
                barrier,
                inc=1,
                device_id=((my_i + k) % N_DEV,),
                device_id_type=pl.DeviceIdType.MESH,
            )

        acc_ref[pl.ds(0, 1), :] = jnp.sum(
            x_ref[...].astype(jnp.float32), axis=0, keepdims=True
        )

        pl.semaphore_wait(barrier, N_DEV - 1)

        sends = []
        for k in range(1, N_DEV):
            rdma = pltpu.make_async_remote_copy(
                src_ref=acc_ref.at[pl.ds(0, 1)],
                dst_ref=acc_ref.at[pl.ds(N_DEV - k, 1)],
                send_sem=send_sems.at[k],
                recv_sem=recv_sems.at[N_DEV - k],
                device_id=((my_i + k) % N_DEV,),
                device_id_type=pl.DeviceIdType.MESH,
            )
            rdma.start()
            sends.append(rdma)

        for j in range(1, N_DEV):
            recv = pltpu.make_async_remote_copy(
                src_ref=acc_ref.at[pl.ds(0, 1)],
                dst_ref=acc_ref.at[pl.ds(j, 1)],
                send_sem=send_sems.at[0],
                recv_sem=recv_sems.at[j],
                device_id=((my_i + 1) % N_DEV,),
                device_id_type=pl.DeviceIdType.MESH,
            )
            recv.wait_recv()

        out_ref[...] = (
            jnp.sum(acc_ref[...], axis=0, keepdims=True) * (1.0 / total_rows)
        ).astype(jnp.float32)

        for rdma in sends:
            rdma.wait_send()

    return pl.pallas_call(
        body,
        out_shape=jax.ShapeDtypeStruct((1, n), jnp.float32),
        in_specs=[pl.BlockSpec(memory_space=pltpu.VMEM)],
        out_specs=pl.BlockSpec(memory_space=pltpu.VMEM),
        scratch_shapes=[
            pltpu.VMEM((N_DEV, n), jnp.float32),
            pltpu.SemaphoreType.DMA((N_DEV,)),
            pltpu.SemaphoreType.DMA((N_DEV,)),
        ],
        compiler_params=pltpu.CompilerParams(collective_id=0),
    )(x)
